# baseline (speedup 1.0000x reference)
import functools

import jax
import jax.numpy as jnp
from jax import lax
from jax.experimental import pallas as pl
from jax.experimental.pallas import tpu as pltpu
from jax.experimental.pallas import tpu_sc as plsc

B, N, KNN = 8, 2048, 20
T = B * N
EPS = 1e-5
F32 = jnp.float32
BF16 = jnp.bfloat16


def _leaky(v):
    return jnp.where(v >= 0, v, 0.2 * v)


_RB = 256


def _knn_body(xall_ref, xrow_ref, idx_ref):
    b = pl.program_id(0)
    xb = xall_ref[0]
    xr = xrow_ref[0]
    sq_all = jnp.sum(xb * xb, axis=1).reshape(1, N)
    sq_r = jnp.sum(xr * xr, axis=1, keepdims=True)
    dot = lax.dot_general(xr.astype(BF16), xb.astype(BF16),
                          (((1,), (1,)), ((), ())),
                          preferred_element_type=F32)
    pd = (2.0 * dot - sq_r) - sq_all
    cols = lax.broadcasted_iota(jnp.int32, (_RB, N), 1)
    picks = []
    for _ in range(KNN):
        m = jnp.max(pd, axis=1, keepdims=True)
        idx = jnp.min(jnp.where(pd == m, cols, N), axis=1, keepdims=True)
        picks.append(idx)
        pd = jnp.where(cols == idx, -jnp.inf, pd)
    idx_ref[0] = jnp.concatenate(picks, axis=1) + b * N


def _knn_indices(xT):
    C = xT.shape[-1]
    return pl.pallas_call(
        _knn_body,
        grid=(B, N // _RB),
        in_specs=[pl.BlockSpec((1, N, C), lambda b, r: (b, 0, 0)),
                  pl.BlockSpec((1, _RB, C), lambda b, r: (b, r, 0))],
        out_specs=pl.BlockSpec((1, _RB, KNN), lambda b, r: (b, r, 0)),
        out_shape=jax.ShapeDtypeStruct((B, N, KNN), jnp.int32),
    )(xT, xT)


_NW = 32
_CH = 128
_CP = 128


@functools.cache
def _make_gather():
    E = KNN * T // _NW
    mesh = plsc.VectorSubcoreMesh(core_axis_name="c", subcore_axis_name="s")

    @functools.partial(
        pl.kernel, mesh=mesh,
        out_type=jax.ShapeDtypeStruct((KNN * T, _CP), F32),
        scratch_types=[pltpu.VMEM((_CH,), jnp.int32),
                       pltpu.VMEM((_CH, _CP), F32),
                       pltpu.SemaphoreType.DMA],
    )
    def g(xf_hbm, idxk_hbm, out_hbm, idx_v, rows_v, sem):
        wid = lax.axis_index("s") * 2 + lax.axis_index("c")

        def chunk(it, carry):
            e0 = pl.multiple_of(wid * E + it * _CH, 8)
            pltpu.sync_copy(idxk_hbm.at[pl.ds(e0, _CH)], idx_v)
            pltpu.async_copy(xf_hbm.at[idx_v], rows_v, sem).wait()
            pltpu.sync_copy(rows_v, out_hbm.at[pl.ds(e0, _CH)])
            return carry

        lax.fori_loop(0, E // _CH, chunk, 0)

    return g


_PB = 512


def _edge_body(xg_ref, xf_ref, w_ref, y_ref):
    C = xf_ref.shape[-1]
    ctr = xf_ref[...]
    ctrb = ctr.astype(BF16)
    wb = w_ref[...].astype(BF16)
    for k in range(KNN):
        nbr = xg_ref[k, :, :C]
        f = jnp.concatenate([(nbr - ctr).astype(BF16), ctrb], axis=1)
        y_ref[k] = jnp.dot(f, wb, preferred_element_type=F32)


def _edge_conv_tc(xg, xf, wT):
    C = xf.shape[-1]
    o = wT.shape[-1]
    return pl.pallas_call(
        _edge_body,
        grid=(T // _PB,),
        in_specs=[pl.BlockSpec((KNN, _PB, _CP), lambda i: (0, i, 0)),
                  pl.BlockSpec((_PB, C), lambda i: (i, 0)),
                  pl.BlockSpec((2 * C, o), lambda i: (0, 0))],
        out_specs=pl.BlockSpec((KNN, _PB, o), lambda i: (0, i, 0)),
        out_shape=jax.ShapeDtypeStruct((KNN, T, o), F32),
    )(xg, xf, wT)


def _edge_conv(xT, W, gam, bet):
    C = xT.shape[-1]
    o = W.shape[0]
    gidx = _knn_indices(xT)
    idxk = jnp.transpose(gidx.reshape(T, KNN)).reshape(KNN * T)
    xf = xT.reshape(T, C)
    xfp = jnp.pad(xf, ((0, 0), (0, _CP - C))) if C < _CP else xf
    xg = _make_gather()(xfp, idxk).reshape(KNN, T, _CP)
    yfull = _edge_conv_tc(xg, xf, jnp.transpose(W))
    y4d = lax.optimization_barrier(
        jnp.transpose(yfull.reshape(KNN, B, N, o), (1, 3, 2, 0)))
    m = jnp.mean(y4d, axis=(0, 2, 3), keepdims=True)[..., 0]
    v = jnp.var(y4d, axis=(0, 2, 3), keepdims=True)[..., 0]
    ymax = jnp.max(y4d, axis=3)
    shp = (1, -1, 1)
    xn = gam.reshape(shp) * (ymax - m) / jnp.sqrt(v + EPS) + bet.reshape(shp)
    xn = jnp.where(xn >= 0, xn, 0.2 * xn)
    return jnp.transpose(xn, (0, 2, 1))


_CB = 2048


def _conv_body(x1, x2, x3, x4, wc_ref, y_ref, s1_ref, s2_ref):
    i = pl.program_id(0)
    xc = jnp.concatenate([x1[...], x2[...], x3[...], x4[...]], axis=1)
    y = jnp.dot(xc.astype(BF16), wc_ref[...].astype(BF16),
                preferred_element_type=F32)
    y_ref[...] = y
    p1 = jnp.sum(y, axis=0, keepdims=True)
    p2 = jnp.sum(y * y, axis=0, keepdims=True)

    @pl.when(i == 0)
    def _():
        s1_ref[...] = p1
        s2_ref[...] = p2

    @pl.when(i > 0)
    def _():
        s1_ref[...] += p1
        s2_ref[...] += p2


def _final_conv(x1, x2, x3, x4, wcT):
    return pl.pallas_call(
        _conv_body,
        grid=(T // _CB,),
        in_specs=[pl.BlockSpec((_CB, 64), lambda i: (i, 0)),
                  pl.BlockSpec((_CB, 64), lambda i: (i, 0)),
                  pl.BlockSpec((_CB, 128), lambda i: (i, 0)),
                  pl.BlockSpec((_CB, 256), lambda i: (i, 0)),
                  pl.BlockSpec((512, 1024), lambda i: (0, 0))],
        out_specs=[pl.BlockSpec((_CB, 1024), lambda i: (i, 0)),
                   pl.BlockSpec((1, 1024), lambda i: (0, 0)),
                   pl.BlockSpec((1, 1024), lambda i: (0, 0))],
        out_shape=[jax.ShapeDtypeStruct((T, 1024), F32),
                   jax.ShapeDtypeStruct((1, 1024), F32),
                   jax.ShapeDtypeStruct((1, 1024), F32)],
    )(x1, x2, x3, x4, wcT)


def _pool_body(s1_ref, s2_ref, g_ref, bt_ref, y_ref, h_ref):
    cnt = float(T)
    m = s1_ref[...] / cnt
    var = s2_ref[...] / cnt - m * m
    scale = g_ref[...] * lax.rsqrt(var + EPS)
    shift = bt_ref[...] - m * scale
    ln = _leaky(y_ref[0] * scale + shift)
    x5 = jnp.max(ln, axis=0, keepdims=True)
    x6 = jnp.sum(ln, axis=0, keepdims=True) * (1.0 / N)
    h_ref[0] = jnp.concatenate([x5, x6], axis=1)


def _pool(s1, s2, gamc, betc, y):
    return pl.pallas_call(
        _pool_body,
        grid=(B,),
        in_specs=[pl.BlockSpec((1, 1024), lambda b: (0, 0))] * 4 +
                 [pl.BlockSpec((1, N, 1024), lambda b: (b, 0, 0))],
        out_specs=pl.BlockSpec((1, 1, 2048), lambda b: (b, 0, 0)),
        out_shape=jax.ShapeDtypeStruct((B, 1, 2048), F32),
    )(s1, s2, gamc.reshape(1, 1024), betc.reshape(1, 1024),
      y.reshape(B, N, 1024)).reshape(B, 2048)


def _bdot(a, b):
    return jnp.dot(a.astype(BF16), b.astype(BF16), preferred_element_type=F32)


def _head_body(h_ref, w1_ref, g1_ref, b1_ref, w2_ref, bb2_ref, g2_ref,
               b2_ref, w3_ref, bb3_ref, out_ref):
    t = _bdot(h_ref[...], w1_ref[...])
    m = jnp.mean(t, axis=0, keepdims=True)
    v = jnp.mean((t - m) ** 2, axis=0, keepdims=True)
    t = _leaky(g1_ref[...] * (t - m) * lax.rsqrt(v + EPS) + b1_ref[...])
    t = _bdot(t, w2_ref[...]) + bb2_ref[...]
    m = jnp.mean(t, axis=0, keepdims=True)
    v = jnp.mean((t - m) ** 2, axis=0, keepdims=True)
    t = _leaky(g2_ref[...] * (t - m) * lax.rsqrt(v + EPS) + b2_ref[...])
    out_ref[...] = _bdot(t, w3_ref[...]) + bb3_ref[...]


def _head(h, Wf1, gamf1, betf1, Wf2, bf2, gamf2, betf2, Wf3, bf3):
    full = lambda s: pl.BlockSpec(s, lambda: tuple(0 for _ in s))
    return pl.pallas_call(
        _head_body,
        grid=(),
        in_specs=[full((B, 2048)), full((2048, 512)), full((1, 512)),
                  full((1, 512)), full((512, 256)), full((1, 256)),
                  full((1, 256)), full((1, 256)), full((256, 40)),
                  full((1, 40))],
        out_specs=full((B, 40)),
        out_shape=jax.ShapeDtypeStruct((B, 40), F32),
    )(h, Wf1.T, gamf1.reshape(1, 512), betf1.reshape(1, 512), Wf2.T,
      bf2.reshape(1, 256), gamf2.reshape(1, 256), betf2.reshape(1, 256),
      Wf3.T, bf3.reshape(1, 40))


def kernel(x, W1, gam1, bet1, W2, gam2, bet2, W3, gam3, bet3, W4, gam4,
           bet4, Wc, gamc, betc, Wf1, gamf1, betf1, Wf2, bf2, gamf2, betf2,
           Wf3, bf3):
    xT = jnp.transpose(x, (0, 2, 1))
    x1 = _edge_conv(xT, W1, gam1, bet1)
    x2 = _edge_conv(x1, W2, gam2, bet2)
    x3 = _edge_conv(x2, W3, gam3, bet3)
    x4 = _edge_conv(x3, W4, gam4, bet4)
    y, s1, s2 = _final_conv(x1.reshape(T, 64), x2.reshape(T, 64),
                            x3.reshape(T, 128), x4.reshape(T, 256),
                            jnp.transpose(Wc))
    h = _pool(s1, s2, gamc, betc, y)
    return _head(h, Wf1, gamf1, betf1, Wf2, bf2, gamf2, betf2, Wf3, bf3)


# speedup vs baseline: 1.1711x; 1.1711x over previous
import functools

import jax
import jax.numpy as jnp
from jax import lax
from jax.experimental import pallas as pl
from jax.experimental.pallas import tpu as pltpu
from jax.experimental.pallas import tpu_sc as plsc

B, N, KNN = 8, 2048, 20
T = B * N
EPS = 1e-5
F32 = jnp.float32
BF16 = jnp.bfloat16


def _leaky(v):
    return jnp.where(v >= 0, v, 0.2 * v)


_RB = 256


def _knn_body(xall_ref, xrow_ref, idx_ref):
    b = pl.program_id(0)
    xb = xall_ref[0]
    xr = xrow_ref[0]
    sq_all = jnp.sum(xb * xb, axis=1).reshape(1, N)
    sq_r = jnp.sum(xr * xr, axis=1, keepdims=True)
    dot = lax.dot_general(xr.astype(BF16), xb.astype(BF16),
                          (((1,), (1,)), ((), ())),
                          preferred_element_type=F32)
    pd = (2.0 * dot - sq_r) - sq_all
    cols = lax.broadcasted_iota(jnp.int32, (_RB, N), 1)
    picks = []
    for _ in range(KNN):
        m = jnp.max(pd, axis=1, keepdims=True)
        idx = jnp.min(jnp.where(pd == m, cols, N), axis=1, keepdims=True)
        picks.append(idx)
        pd = jnp.where(cols == idx, -jnp.inf, pd)
    idx_ref[0] = jnp.concatenate(picks, axis=1) + b * N


def _knn_indices(xT):
    C = xT.shape[-1]
    return pl.pallas_call(
        _knn_body,
        grid=(B, N // _RB),
        in_specs=[pl.BlockSpec((1, N, C), lambda b, r: (b, 0, 0)),
                  pl.BlockSpec((1, _RB, C), lambda b, r: (b, r, 0))],
        out_specs=pl.BlockSpec((1, _RB, KNN), lambda b, r: (b, r, 0)),
        out_shape=jax.ShapeDtypeStruct((B, N, KNN), jnp.int32),
    )(xT, xT)


_NW = 32
_CH = 128
_CP = 128


@functools.cache
def _make_gather():
    E = KNN * T // _NW
    mesh = plsc.VectorSubcoreMesh(core_axis_name="c", subcore_axis_name="s")

    @functools.partial(
        pl.kernel, mesh=mesh,
        out_type=jax.ShapeDtypeStruct((KNN * T, _CP), F32),
        scratch_types=[pltpu.VMEM((_CH,), jnp.int32),
                       pltpu.VMEM((_CH, _CP), F32),
                       pltpu.SemaphoreType.DMA],
    )
    def g(xf_hbm, idxk_hbm, out_hbm, idx_v, rows_v, sem):
        wid = lax.axis_index("s") * 2 + lax.axis_index("c")

        def chunk(it, carry):
            e0 = pl.multiple_of(wid * E + it * _CH, 8)
            pltpu.sync_copy(idxk_hbm.at[pl.ds(e0, _CH)], idx_v)
            pltpu.async_copy(xf_hbm.at[idx_v], rows_v, sem).wait()
            pltpu.sync_copy(rows_v, out_hbm.at[pl.ds(e0, _CH)])
            return carry

        lax.fori_loop(0, E // _CH, chunk, 0)

    return g


_PB = 512


def _edge_body(xg_ref, xf_ref, w_ref, ymax_ref, s1_ref, s2_ref):
    i = pl.program_id(0)
    C = xf_ref.shape[-1]
    ctr = xf_ref[...]
    ctrb = ctr.astype(BF16)
    wb = w_ref[...].astype(BF16)
    acc = None
    p1 = None
    p2 = None
    for k in range(KNN):
        nbr = xg_ref[k, :, :C]
        f = jnp.concatenate([(nbr - ctr).astype(BF16), ctrb], axis=1)
        y = jnp.dot(f, wb, preferred_element_type=F32)
        acc = y if acc is None else jnp.maximum(acc, y)
        q1 = jnp.sum(y, axis=0, keepdims=True)
        q2 = jnp.sum(y * y, axis=0, keepdims=True)
        p1 = q1 if p1 is None else p1 + q1
        p2 = q2 if p2 is None else p2 + q2
    ymax_ref[...] = acc

    @pl.when(i == 0)
    def _():
        s1_ref[...] = p1
        s2_ref[...] = p2

    @pl.when(i > 0)
    def _():
        s1_ref[...] += p1
        s2_ref[...] += p2


def _edge_conv_tc(xg, xf, wT):
    C = xf.shape[-1]
    o = wT.shape[-1]
    return pl.pallas_call(
        _edge_body,
        grid=(T // _PB,),
        in_specs=[pl.BlockSpec((KNN, _PB, _CP), lambda i: (0, i, 0)),
                  pl.BlockSpec((_PB, C), lambda i: (i, 0)),
                  pl.BlockSpec((2 * C, o), lambda i: (0, 0))],
        out_specs=[pl.BlockSpec((_PB, o), lambda i: (i, 0)),
                   pl.BlockSpec((1, o), lambda i: (0, 0)),
                   pl.BlockSpec((1, o), lambda i: (0, 0))],
        out_shape=[jax.ShapeDtypeStruct((T, o), F32),
                   jax.ShapeDtypeStruct((1, o), F32),
                   jax.ShapeDtypeStruct((1, o), F32)],
    )(xg, xf, wT)


_SB = 2048


def _norm_body(s1_ref, s2_ref, g_ref, bt_ref, ym_ref, out_ref):
    cnt = float(T * KNN)
    m = s1_ref[...] / cnt
    var = s2_ref[...] / cnt - m * m
    den = jnp.sqrt(var + EPS)
    out_ref[...] = _leaky(g_ref[...] * (ym_ref[...] - m) / den + bt_ref[...])


def _edge_norm(s1, s2, gam, bet, ym):
    o = ym.shape[-1]
    return pl.pallas_call(
        _norm_body,
        grid=(T // _SB,),
        in_specs=[pl.BlockSpec((1, o), lambda i: (0, 0))] * 4 +
                 [pl.BlockSpec((_SB, o), lambda i: (i, 0))],
        out_specs=pl.BlockSpec((_SB, o), lambda i: (i, 0)),
        out_shape=jax.ShapeDtypeStruct((T, o), F32),
    )(s1, s2, gam.reshape(1, o), bet.reshape(1, o), ym)


def _edge_conv(xT, W, gam, bet):
    C = xT.shape[-1]
    o = W.shape[0]
    gidx = _knn_indices(xT)
    idxk = jnp.transpose(gidx.reshape(T, KNN)).reshape(KNN * T)
    xf = xT.reshape(T, C)
    xfp = jnp.pad(xf, ((0, 0), (0, _CP - C))) if C < _CP else xf
    xg = _make_gather()(xfp, idxk).reshape(KNN, T, _CP)
    ymax, s1, s2 = _edge_conv_tc(xg, xf, jnp.transpose(W))
    out = _edge_norm(s1, s2, gam, bet, ymax)
    return out.reshape(B, N, o)


_CB = 2048


def _conv_body(x1, x2, x3, x4, wc_ref, y_ref, s1_ref, s2_ref):
    i = pl.program_id(0)
    xc = jnp.concatenate([x1[...], x2[...], x3[...], x4[...]], axis=1)
    y = jnp.dot(xc.astype(BF16), wc_ref[...].astype(BF16),
                preferred_element_type=F32)
    y_ref[...] = y
    p1 = jnp.sum(y, axis=0, keepdims=True)
    p2 = jnp.sum(y * y, axis=0, keepdims=True)

    @pl.when(i == 0)
    def _():
        s1_ref[...] = p1
        s2_ref[...] = p2

    @pl.when(i > 0)
    def _():
        s1_ref[...] += p1
        s2_ref[...] += p2


def _final_conv(x1, x2, x3, x4, wcT):
    return pl.pallas_call(
        _conv_body,
        grid=(T // _CB,),
        in_specs=[pl.BlockSpec((_CB, 64), lambda i: (i, 0)),
                  pl.BlockSpec((_CB, 64), lambda i: (i, 0)),
                  pl.BlockSpec((_CB, 128), lambda i: (i, 0)),
                  pl.BlockSpec((_CB, 256), lambda i: (i, 0)),
                  pl.BlockSpec((512, 1024), lambda i: (0, 0))],
        out_specs=[pl.BlockSpec((_CB, 1024), lambda i: (i, 0)),
                   pl.BlockSpec((1, 1024), lambda i: (0, 0)),
                   pl.BlockSpec((1, 1024), lambda i: (0, 0))],
        out_shape=[jax.ShapeDtypeStruct((T, 1024), F32),
                   jax.ShapeDtypeStruct((1, 1024), F32),
                   jax.ShapeDtypeStruct((1, 1024), F32)],
    )(x1, x2, x3, x4, wcT)


def _pool_body(s1_ref, s2_ref, g_ref, bt_ref, y_ref, h_ref):
    cnt = float(T)
    m = s1_ref[...] / cnt
    var = s2_ref[...] / cnt - m * m
    scale = g_ref[...] * lax.rsqrt(var + EPS)
    shift = bt_ref[...] - m * scale
    ln = _leaky(y_ref[0] * scale + shift)
    x5 = jnp.max(ln, axis=0, keepdims=True)
    x6 = jnp.sum(ln, axis=0, keepdims=True) * (1.0 / N)
    h_ref[0] = jnp.concatenate([x5, x6], axis=1)


def _pool(s1, s2, gamc, betc, y):
    return pl.pallas_call(
        _pool_body,
        grid=(B,),
        in_specs=[pl.BlockSpec((1, 1024), lambda b: (0, 0))] * 4 +
                 [pl.BlockSpec((1, N, 1024), lambda b: (b, 0, 0))],
        out_specs=pl.BlockSpec((1, 1, 2048), lambda b: (b, 0, 0)),
        out_shape=jax.ShapeDtypeStruct((B, 1, 2048), F32),
    )(s1, s2, gamc.reshape(1, 1024), betc.reshape(1, 1024),
      y.reshape(B, N, 1024)).reshape(B, 2048)


def _bdot(a, b):
    return jnp.dot(a.astype(BF16), b.astype(BF16), preferred_element_type=F32)


def _head_body(h_ref, w1_ref, g1_ref, b1_ref, w2_ref, bb2_ref, g2_ref,
               b2_ref, w3_ref, bb3_ref, out_ref):
    t = _bdot(h_ref[...], w1_ref[...])
    m = jnp.mean(t, axis=0, keepdims=True)
    v = jnp.mean((t - m) ** 2, axis=0, keepdims=True)
    t = _leaky(g1_ref[...] * (t - m) * lax.rsqrt(v + EPS) + b1_ref[...])
    t = _bdot(t, w2_ref[...]) + bb2_ref[...]
    m = jnp.mean(t, axis=0, keepdims=True)
    v = jnp.mean((t - m) ** 2, axis=0, keepdims=True)
    t = _leaky(g2_ref[...] * (t - m) * lax.rsqrt(v + EPS) + b2_ref[...])
    out_ref[...] = _bdot(t, w3_ref[...]) + bb3_ref[...]


def _head(h, Wf1, gamf1, betf1, Wf2, bf2, gamf2, betf2, Wf3, bf3):
    full = lambda s: pl.BlockSpec(s, lambda: tuple(0 for _ in s))
    return pl.pallas_call(
        _head_body,
        grid=(),
        in_specs=[full((B, 2048)), full((2048, 512)), full((1, 512)),
                  full((1, 512)), full((512, 256)), full((1, 256)),
                  full((1, 256)), full((1, 256)), full((256, 40)),
                  full((1, 40))],
        out_specs=full((B, 40)),
        out_shape=jax.ShapeDtypeStruct((B, 40), F32),
    )(h, Wf1.T, gamf1.reshape(1, 512), betf1.reshape(1, 512), Wf2.T,
      bf2.reshape(1, 256), gamf2.reshape(1, 256), betf2.reshape(1, 256),
      Wf3.T, bf3.reshape(1, 40))


def kernel(x, W1, gam1, bet1, W2, gam2, bet2, W3, gam3, bet3, W4, gam4,
           bet4, Wc, gamc, betc, Wf1, gamf1, betf1, Wf2, bf2, gamf2, betf2,
           Wf3, bf3):
    xT = jnp.transpose(x, (0, 2, 1))
    x1 = _edge_conv(xT, W1, gam1, bet1)
    x2 = _edge_conv(x1, W2, gam2, bet2)
    x3 = _edge_conv(x2, W3, gam3, bet3)
    x4 = _edge_conv(x3, W4, gam4, bet4)
    y, s1, s2 = _final_conv(x1.reshape(T, 64), x2.reshape(T, 64),
                            x3.reshape(T, 128), x4.reshape(T, 256),
                            jnp.transpose(Wc))
    h = _pool(s1, s2, gamc, betc, y)
    return _head(h, Wf1, gamf1, betf1, Wf2, bf2, gamf2, betf2, Wf3, bf3)
